# baseline (speedup 1.0000x reference)
import functools

import jax
import jax.numpy as jnp
from jax import lax
from jax.experimental import pallas as pl
from jax.experimental.pallas import tpu as pltpu
from jax.experimental.pallas import tpu_sc as plsc

B, N, D = 32, 256, 384
CODE_DIM, N_CODES, N_SLOTS = 64, 8192, 8


def _idx_kernel(vmask_ref, V_ref, xn_ref, ln1_g_ref, ln1_b_ref, W1_ref, b1_ref,
                W2_ref, b2_ref, mu_ref, idx_ref, proj_ref):
    g = pl.program_id(0)
    gsz = B // 4
    r8r = lax.broadcasted_iota(jnp.int32, (8, 8), 0)
    r8c = lax.broadcasted_iota(jnp.int32, (8, 8), 1)
    R8 = (r8r + r8c == 7).astype(jnp.float32)

    def body(bi, _):
        Vc = lax.dot_general(V_ref[bi, :, 120:], R8, (((1,), (0,)), ((), ())),
                             preferred_element_type=jnp.float32)
        frac_pos = jnp.mean((Vc > 0).astype(jnp.float32), axis=0, keepdims=True)
        sign = jnp.where(frac_pos < 0.5, -1.0, 1.0)
        scale = sign * vmask_ref[...]
        proj_ref[pl.ds((g * gsz + bi) * 8, 8), :] = lax.dot_general(
            Vc * scale, xn_ref[bi], (((0,), (0,)), ((), ())),
            preferred_element_type=jnp.float32)
        return 0

    lax.fori_loop(0, gsz, body, 0, unroll=True)

    @pl.when(g == 3)
    def _tail():
        _finish(proj_ref, ln1_g_ref, ln1_b_ref, W1_ref, b1_ref, W2_ref,
                b2_ref, mu_ref, idx_ref)


def _finish(proj_ref, ln1_g_ref, ln1_b_ref, W1_ref, b1_ref, W2_ref, b2_ref,
            mu_ref, idx_ref):
    proj = proj_ref[...]
    m = jnp.mean(proj, axis=-1, keepdims=True)
    v = jnp.mean((proj - m) ** 2, axis=-1, keepdims=True)
    h = (proj - m) / jnp.sqrt(v + 1e-5) * ln1_g_ref[...] + ln1_b_ref[...]
    h = jnp.maximum(
        lax.dot_general(h, W1_ref[...], (((1,), (0,)), ((), ())),
                        preferred_element_type=jnp.float32) + b1_ref[...], 0.0)
    z = lax.dot_general(h, W2_ref[...], (((1,), (0,)), ((), ())),
                        preferred_element_type=jnp.float32) + b2_ref[...]
    zn = z / jnp.maximum(jnp.sqrt(jnp.sum(z * z, axis=-1, keepdims=True)), 1e-8)
    mu = mu_ref[...]
    norm = jnp.sqrt(jnp.sum(mu * mu, axis=1, keepdims=True))
    mun = mu / jnp.maximum(norm, 1e-8)
    sim = lax.dot_general(zn, mun, (((1,), (1,)), ((), ())),
                          preferred_element_type=jnp.float32)
    idx_ref[...] = jnp.argmax(sim, axis=-1).astype(jnp.int32).reshape(1, 256)


def _up_kernel(rows_ref, noise_ref, W3_ref, b3_ref, W4_ref,
               b4_ref, ln2_g_ref, ln2_b_ref, out_ref):
    rows = rows_ref[...]
    sample = rows[:, :CODE_DIM] + jnp.exp(rows[:, CODE_DIM:]) * noise_ref[...]
    u = jnp.maximum(
        lax.dot_general(sample, W3_ref[...], (((1,), (0,)), ((), ())),
                        preferred_element_type=jnp.float32) + b3_ref[...], 0.0)
    u = lax.dot_general(u, W4_ref[...], (((1,), (0,)), ((), ())),
                        preferred_element_type=jnp.float32) + b4_ref[...]
    m = jnp.mean(u, axis=-1, keepdims=True)
    v = jnp.mean((u - m) ** 2, axis=-1, keepdims=True)
    out_ref[...] = (u - m) / jnp.sqrt(v + 1e-5) * ln2_g_ref[...] + ln2_b_ref[...]


def _sc_gather(table, idx_flat):
    width = table.shape[1]
    nrows = idx_flat.shape[0]
    info = plsc.get_sparse_core_info()
    nc, ns = info.num_cores, info.num_subcores
    rows_per_w = nrows // (nc * ns)
    mesh = plsc.VectorSubcoreMesh(core_axis_name="c", subcore_axis_name="s")

    @functools.partial(
        pl.kernel, mesh=mesh,
        out_type=jax.ShapeDtypeStruct((nrows, width), jnp.float32),
        scratch_types=[
            pltpu.VMEM((rows_per_w,), jnp.int32),
            pltpu.VMEM((rows_per_w, width), jnp.float32),
            pltpu.SemaphoreType.DMA,
        ],
    )
    def k(table_hbm, idx_hbm, out_hbm, idx_v, rows_v, sem):
        wid = lax.axis_index("s") * nc + lax.axis_index("c")
        base = wid * rows_per_w
        pltpu.sync_copy(idx_hbm.at[pl.ds(base, rows_per_w)], idx_v)
        pltpu.async_copy(table_hbm.at[idx_v], rows_v, sem).wait()
        pltpu.sync_copy(rows_v, out_hbm.at[pl.ds(base, rows_per_w)])

    return k(table, idx_flat)


def kernel(x, n_slots, mu, log_sigma, ln1_g, ln1_b, W1, b1, W2, b2, W3, b3,
           W4, b4, ln2_g, ln2_b, noise):
    xn = x / jnp.maximum(jnp.linalg.norm(x, axis=-1, keepdims=True), 1e-12)
    cov = jnp.einsum('bnd,bmd->bnm', xn, xn)
    _, eig_vectors = jnp.linalg.eigh(cov)

    slots = noise.shape[1]
    vmask = (jnp.arange(slots) < n_slots).astype(jnp.float32).reshape(1, slots)
    idx = pl.pallas_call(
        _idx_kernel,
        grid=(4,),
        in_specs=[
            pl.BlockSpec((1, slots), lambda g: (0, 0)),
            pl.BlockSpec((B // 4, N, 128), lambda g: (g, 0, N // 128 - 1)),
            pl.BlockSpec((B // 4, N, D), lambda g: (g, 0, 0)),
            pl.BlockSpec((1, D), lambda g: (0, 0)),
            pl.BlockSpec((1, D), lambda g: (0, 0)),
            pl.BlockSpec((D, D), lambda g: (0, 0)),
            pl.BlockSpec((1, D), lambda g: (0, 0)),
            pl.BlockSpec((D, CODE_DIM), lambda g: (0, 0)),
            pl.BlockSpec((1, CODE_DIM), lambda g: (0, 0)),
            pl.BlockSpec((N_CODES, CODE_DIM), lambda g: (0, 0)),
        ],
        out_specs=pl.BlockSpec((1, B * slots), lambda g: (0, 0)),
        out_shape=jax.ShapeDtypeStruct((1, B * slots), jnp.int32),
        scratch_shapes=[pltpu.VMEM((B * slots, D), jnp.float32)],
    )(vmask, eig_vectors, xn, ln1_g.reshape(1, D), ln1_b.reshape(1, D),
      W1, b1.reshape(1, D), W2, b2.reshape(1, CODE_DIM), mu)

    idx_flat = idx.reshape(B * slots)

    rows = _sc_gather(jnp.concatenate([mu, log_sigma], axis=1), idx_flat)

    out = pl.pallas_call(
        _up_kernel,
        in_specs=[
            pl.BlockSpec((B * slots, 2 * CODE_DIM), lambda: (0, 0)),
            pl.BlockSpec((B * slots, CODE_DIM), lambda: (0, 0)),
            pl.BlockSpec((CODE_DIM, D), lambda: (0, 0)),
            pl.BlockSpec((1, D), lambda: (0, 0)),
            pl.BlockSpec((D, D), lambda: (0, 0)),
            pl.BlockSpec((1, D), lambda: (0, 0)),
            pl.BlockSpec((1, D), lambda: (0, 0)),
            pl.BlockSpec((1, D), lambda: (0, 0)),
        ],
        out_specs=pl.BlockSpec((B * slots, D), lambda: (0, 0)),
        out_shape=jax.ShapeDtypeStruct((B * slots, D), jnp.float32),
    )(rows, noise.reshape(B * slots, CODE_DIM), W3,
      b3.reshape(1, D), W4, b4.reshape(1, D), ln2_g.reshape(1, D),
      ln2_b.reshape(1, D))

    return out.reshape(B, slots, D)
